# baseline (speedup 1.0000x reference)
import functools

import jax
import jax.numpy as jnp
from jax import lax
from jax.experimental import pallas as pl
from jax.experimental.pallas import tpu as pltpu
from jax.experimental.pallas import tpu_sc as plsc

N = 1024
D = 64
D_USE = 48
M = 1000
MPAD = 1024
EPS = 1e-8
NPAIRS = N * (N - 1) // 2

_NC = 2
_NS = 16
_NW = _NC * _NS
_ROWS = N // _NW
_L = 16


def _sc_affinity_body(yt_hbm, lut_hbm, a_hbm, yt_v, sidx_v, rows_v,
                      out_v, sem, sem2):
    wid = lax.axis_index("s") * _NC + lax.axis_index("c")
    base = pl.multiple_of(wid * _ROWS, _ROWS)
    pltpu.sync_copy(yt_hbm, yt_v)
    lane = lax.iota(jnp.int32, _L)
    for v in range(_ROWS * 8 // _L):
        j = lane + (v * _L)
        yt16 = plsc.load_gather(yt_v, [base + (j >> 3)])
        sidx_v[v // 8, pl.ds((v % 8) * _L, _L)] = (yt16 << 3) + (j & 7)
    cp0 = pltpu.async_copy(
        lut_hbm.at[sidx_v.at[0]], rows_v.at[pl.ds(0, 128)], sem)
    cp1 = pltpu.async_copy(
        lut_hbm.at[sidx_v.at[1]], rows_v.at[pl.ds(128, 128)], sem)
    cp0.wait()
    cp1.wait()

    def make_g_body(rlo):
        def g_body(g, _):
            gbase = pl.multiple_of(g * _L, _L)
            col16 = yt_v[pl.ds(gbase, _L)]
            srow = col16 >> 7
            lcol = col16 & 127
            sub = g // 8
            off = pl.multiple_of((g % 8) * _L, _L)
            for r0 in range(rlo, rlo + _ROWS // 2, 8):
                vals = [
                    plsc.load_gather(rows_v, [srow + (r0 + k) * 8, lcol])
                    for k in range(8)
                ]
                for k in range(8):
                    out_v[(r0 + k) * 8 + sub, pl.ds(off, _L)] = vals[k]
            return 0
        return g_body

    lax.fori_loop(0, N // _L, make_g_body(0), 0)
    cph = pltpu.async_copy(
        out_v.at[pl.ds(0, _ROWS * 4)],
        a_hbm.at[pl.ds(base * 8, _ROWS * 4), :], sem2)
    lax.fori_loop(0, N // _L, make_g_body(_ROWS // 2), 0)
    cph.wait()
    pltpu.sync_copy(
        out_v.at[pl.ds(_ROWS * 4, _ROWS * 4)],
        a_hbm.at[pl.ds(base * 8 + _ROWS * 4, _ROWS * 4), :])


def _sc_affinity(y_true, lut_sub):
    mesh = plsc.VectorSubcoreMesh(core_axis_name="c", subcore_axis_name="s")
    fn = pl.kernel(
        _sc_affinity_body,
        out_type=jax.ShapeDtypeStruct((N * 8, N // 8), jnp.float32),
        mesh=mesh,
        scratch_types=[
            pltpu.VMEM((N,), jnp.int32),
            pltpu.VMEM((2, 128), jnp.int32),
            pltpu.VMEM((_ROWS * 8, N // 8), jnp.float32),
            pltpu.VMEM((_ROWS * 8, N // 8), jnp.float32),
            pltpu.SemaphoreType.DMA,
            pltpu.SemaphoreType.DMA,
        ],
        compiler_params=pltpu.CompilerParams(
            use_tc_tiling_on_sc=False, needs_layout_passes=False
        ),
    )
    return fn(y_true, lut_sub)


_RB = 128
_LUTB = 200


def _tc_retile_body(lut_ref, out_ref):
    x = lut_ref[...]
    x = jnp.concatenate(
        [x, jnp.zeros((_LUTB, N - M), jnp.float32)], axis=1)
    out_ref[...] = jnp.reshape(x, (_LUTB * 8, N // 8))


def _tc_retile(lookup):
    grid = M // _LUTB
    return pl.pallas_call(
        _tc_retile_body,
        grid=(grid,),
        in_specs=[pl.BlockSpec((_LUTB, M), lambda i: (i, 0))],
        out_specs=pl.BlockSpec((_LUTB * 8, N // 8), lambda i: (i, 0)),
        out_shape=jax.ShapeDtypeStruct((M * 8, N // 8), jnp.float32),
        compiler_params=pltpu.CompilerParams(
            dimension_semantics=("parallel",),
        ),
    )(lookup)


def _norm_rows(yp):
    colmask = lax.broadcasted_iota(jnp.int32, yp.shape, 1) < D_USE
    ypm = jnp.where(colmask, yp, 0.0)
    sq = jnp.sum(ypm * ypm, axis=1, keepdims=True)
    inv = 1.0 / jnp.maximum(jnp.sqrt(sq), EPS)
    return ypm * inv


def _tc_gram_body(ypb_ref, ypf_ref, g_ref):
    ynb = _norm_rows(ypb_ref[...])
    ynf = _norm_rows(ypf_ref[...])
    g_ref[...] = lax.dot_general(
        ynb, ynf, (((1,), (1,)), ((), ())),
        preferred_element_type=jnp.float32,
        precision=lax.Precision.HIGHEST,
    )


def _tc_gram(y_pred):
    grid = N // _RB
    return pl.pallas_call(
        _tc_gram_body,
        grid=(grid,),
        in_specs=[
            pl.BlockSpec((_RB, D), lambda i: (i, 0)),
            pl.BlockSpec((N, D), lambda i: (0, 0)),
        ],
        out_specs=pl.BlockSpec((_RB, N), lambda i: (i, 0)),
        out_shape=jax.ShapeDtypeStruct((N, N), jnp.float32),
        compiler_params=pltpu.CompilerParams(
            dimension_semantics=("parallel",),
        ),
    )(y_pred, y_pred)


_CB = 512


def _tc_combine_body(g_ref, a_ref, out_ref):
    i = pl.program_id(0)
    a = jnp.reshape(a_ref[...], (_CB, N))
    row = i * _CB + lax.broadcasted_iota(jnp.int32, (_CB, N), 0)
    col = lax.broadcasted_iota(jnp.int32, (_CB, N), 1)
    diff = jnp.where(col > row, jnp.abs(g_ref[...] - a), 0.0)
    s = jnp.sum(diff) * (1.0 / NPAIRS)

    @pl.when(i == 0)
    def _():
        out_ref[...] = jnp.zeros_like(out_ref)

    out_ref[...] += jnp.reshape(s, (1, 1))


def _tc_combine(g_mat, a_t):
    grid = N // _CB
    return pl.pallas_call(
        _tc_combine_body,
        grid=(grid,),
        in_specs=[
            pl.BlockSpec((_CB, N), lambda i: (i, 0)),
            pl.BlockSpec((_CB * 8, N // 8), lambda i: (i, 0)),
        ],
        out_specs=pl.BlockSpec((1, 1), lambda i: (0, 0)),
        out_shape=jax.ShapeDtypeStruct((1, 1), jnp.float32),
        compiler_params=pltpu.CompilerParams(
            dimension_semantics=("arbitrary",),
        ),
    )(g_mat, a_t)


@jax.jit
def kernel(y_true, y_pred, lookup):
    yt = y_true.astype(jnp.int32)
    lut_sub = _tc_retile(lookup)
    a_t = _sc_affinity(yt, lut_sub)
    g_mat = _tc_gram(y_pred)
    return _tc_combine(g_mat, a_t)[0, 0]


# speedup vs baseline: 1.0045x; 1.0045x over previous
import functools

import jax
import jax.numpy as jnp
from jax import lax
from jax.experimental import pallas as pl
from jax.experimental.pallas import tpu as pltpu
from jax.experimental.pallas import tpu_sc as plsc

N = 1024
D = 64
D_USE = 48
M = 1000
MPAD = 1024
EPS = 1e-8
NPAIRS = N * (N - 1) // 2

_NC = 2
_NS = 16
_NW = _NC * _NS
_ROWS = N // _NW
_L = 16


def _sc_affinity_body(yt_hbm, lut_hbm, a_hbm, yt_v, sidx_v, rows_v,
                      out_v, sem, sem2):
    wid = lax.axis_index("s") * _NC + lax.axis_index("c")
    base = pl.multiple_of(wid * _ROWS, _ROWS)
    pltpu.sync_copy(yt_hbm, yt_v)
    lane = lax.iota(jnp.int32, _L)
    for v in range(_ROWS * 8 // _L):
        j = lane + (v * _L)
        yt16 = plsc.load_gather(yt_v, [base + (j >> 3)])
        sidx_v[v // 8, pl.ds((v % 8) * _L, _L)] = (yt16 << 3) + (j & 7)
    cp0 = pltpu.async_copy(
        lut_hbm.at[sidx_v.at[0]], rows_v.at[pl.ds(0, 128)], sem)
    cp1 = pltpu.async_copy(
        lut_hbm.at[sidx_v.at[1]], rows_v.at[pl.ds(128, 128)], sem)
    cp0.wait()
    cp1.wait()

    def g_body(g, _):
        gbase = pl.multiple_of(g * _L, _L)
        col16 = yt_v[pl.ds(gbase, _L)]
        srow = col16 >> 7
        lcol = col16 & 127
        sub = g // 8
        off = pl.multiple_of((g % 8) * _L, _L)
        for r0 in range(0, _ROWS, 8):
            vals = [
                plsc.load_gather(rows_v, [srow + (r0 + k) * 8, lcol])
                for k in range(8)
            ]
            for k in range(8):
                out_v[(r0 + k) * 8 + sub, pl.ds(off, _L)] = vals[k]
        return 0

    lax.fori_loop(0, N // _L, g_body, 0)
    pltpu.sync_copy(out_v, a_hbm.at[pl.ds(base * 8, _ROWS * 8), :])


def _sc_affinity(y_true, lut_sub):
    mesh = plsc.VectorSubcoreMesh(core_axis_name="c", subcore_axis_name="s")
    fn = pl.kernel(
        _sc_affinity_body,
        out_type=jax.ShapeDtypeStruct((N * 8, N // 8), jnp.float32),
        mesh=mesh,
        scratch_types=[
            pltpu.VMEM((N,), jnp.int32),
            pltpu.VMEM((2, 128), jnp.int32),
            pltpu.VMEM((_ROWS * 8, N // 8), jnp.float32),
            pltpu.VMEM((_ROWS * 8, N // 8), jnp.float32),
            pltpu.SemaphoreType.DMA,
            pltpu.SemaphoreType.DMA,
        ],
        compiler_params=pltpu.CompilerParams(
            use_tc_tiling_on_sc=False, needs_layout_passes=False
        ),
    )
    return fn(y_true, lut_sub)


_RB = 128
_LUTB = 200


def _tc_retile_body(lut_ref, out_ref):
    x = lut_ref[...]
    x = jnp.concatenate(
        [x, jnp.zeros((_LUTB, N - M), jnp.float32)], axis=1)
    out_ref[...] = jnp.reshape(x, (_LUTB * 8, N // 8))


def _tc_retile(lookup):
    grid = M // _LUTB
    return pl.pallas_call(
        _tc_retile_body,
        grid=(grid,),
        in_specs=[pl.BlockSpec((_LUTB, M), lambda i: (i, 0))],
        out_specs=pl.BlockSpec((_LUTB * 8, N // 8), lambda i: (i, 0)),
        out_shape=jax.ShapeDtypeStruct((M * 8, N // 8), jnp.float32),
        compiler_params=pltpu.CompilerParams(
            dimension_semantics=("parallel",),
        ),
    )(lookup)


def _norm_rows(yp):
    colmask = lax.broadcasted_iota(jnp.int32, yp.shape, 1) < D_USE
    ypm = jnp.where(colmask, yp, 0.0)
    sq = jnp.sum(ypm * ypm, axis=1, keepdims=True)
    inv = 1.0 / jnp.maximum(jnp.sqrt(sq), EPS)
    return ypm * inv


def _tc_gram_body(ypb_ref, ypf_ref, g_ref):
    ynb = _norm_rows(ypb_ref[...])
    ynf = _norm_rows(ypf_ref[...])
    g_ref[...] = lax.dot_general(
        ynb, ynf, (((1,), (1,)), ((), ())),
        preferred_element_type=jnp.float32,
        precision=lax.Precision.HIGHEST,
    )


def _tc_gram(y_pred):
    grid = N // _RB
    return pl.pallas_call(
        _tc_gram_body,
        grid=(grid,),
        in_specs=[
            pl.BlockSpec((_RB, D), lambda i: (i, 0)),
            pl.BlockSpec((N, D), lambda i: (0, 0)),
        ],
        out_specs=pl.BlockSpec((_RB, N), lambda i: (i, 0)),
        out_shape=jax.ShapeDtypeStruct((N, N), jnp.float32),
        compiler_params=pltpu.CompilerParams(
            dimension_semantics=("parallel",),
        ),
    )(y_pred, y_pred)


_CB = 512


def _tc_combine_body(g_ref, a_ref, out_ref):
    i = pl.program_id(0)
    a = jnp.reshape(a_ref[...], (_CB, N))
    row = i * _CB + lax.broadcasted_iota(jnp.int32, (_CB, N), 0)
    col = lax.broadcasted_iota(jnp.int32, (_CB, N), 1)
    diff = jnp.where(col > row, jnp.abs(g_ref[...] - a), 0.0)
    s = jnp.sum(diff) * (1.0 / NPAIRS)

    @pl.when(i == 0)
    def _():
        out_ref[...] = jnp.zeros_like(out_ref)

    out_ref[...] += jnp.reshape(s, (1, 1))


def _tc_combine(g_mat, a_t):
    grid = N // _CB
    return pl.pallas_call(
        _tc_combine_body,
        grid=(grid,),
        in_specs=[
            pl.BlockSpec((_CB, N), lambda i: (i, 0)),
            pl.BlockSpec((_CB * 8, N // 8), lambda i: (i, 0)),
        ],
        out_specs=pl.BlockSpec((1, 1), lambda i: (0, 0)),
        out_shape=jax.ShapeDtypeStruct((1, 1), jnp.float32),
        compiler_params=pltpu.CompilerParams(
            dimension_semantics=("arbitrary",),
        ),
    )(g_mat, a_t)


@jax.jit
def kernel(y_true, y_pred, lookup):
    yt = y_true.astype(jnp.int32)
    lut_sub = _tc_retile(lookup)
    a_t = _sc_affinity(yt, lut_sub)
    g_mat = _tc_gram(y_pred)
    return _tc_combine(g_mat, a_t)[0, 0]


# speedup vs baseline: 1.0196x; 1.0151x over previous
import functools

import jax
import jax.numpy as jnp
from jax import lax
from jax.experimental import pallas as pl
from jax.experimental.pallas import tpu as pltpu
from jax.experimental.pallas import tpu_sc as plsc

N = 1024
D = 64
D_USE = 48
M = 1000
MPAD = 1024
EPS = 1e-8
NPAIRS = N * (N - 1) // 2

_NC = 2
_NS = 16
_NW = _NC * _NS
_ROWS = N // _NW
_L = 16


def _sc_affinity_body(yt_hbm, lut_hbm, a_hbm, yt_v, idx_v, sidx_v, rows_v,
                      out_v, sem, sem2):
    wid = lax.axis_index("s") * _NC + lax.axis_index("c")
    base = pl.multiple_of(wid * _ROWS, _ROWS)
    pltpu.sync_copy(yt_hbm, yt_v)
    pltpu.sync_copy(yt_hbm.at[pl.ds(base, _ROWS)], idx_v)
    lane = lax.iota(jnp.int32, _L)
    for v in range(_ROWS * 8 // _L):
        j = lane + (v * _L)
        yt16 = plsc.load_gather(idx_v, [j >> 3])
        sidx_v[v // 8, pl.ds((v % 8) * _L, _L)] = (yt16 << 3) + (j & 7)
    cp0 = pltpu.async_copy(
        lut_hbm.at[sidx_v.at[0]], rows_v.at[pl.ds(0, 128)], sem)
    cp1 = pltpu.async_copy(
        lut_hbm.at[sidx_v.at[1]], rows_v.at[pl.ds(128, 128)], sem)
    cp0.wait()
    cp1.wait()

    def g_body(g, _):
        gbase = pl.multiple_of(g * _L, _L)
        col16 = yt_v[pl.ds(gbase, _L)]
        srow = col16 >> 7
        lcol = col16 & 127
        sub = g // 8
        off = pl.multiple_of((g % 8) * _L, _L)
        for r0 in range(0, _ROWS, 8):
            vals = [
                plsc.load_gather(rows_v, [srow + (r0 + k) * 8, lcol])
                for k in range(8)
            ]
            for k in range(8):
                out_v[(r0 + k) * 8 + sub, pl.ds(off, _L)] = vals[k]
        return 0

    lax.fori_loop(0, N // _L, g_body, 0)
    pltpu.sync_copy(out_v, a_hbm.at[pl.ds(base * 8, _ROWS * 8), :])


def _sc_affinity(y_true, lut_sub):
    mesh = plsc.VectorSubcoreMesh(core_axis_name="c", subcore_axis_name="s")
    fn = pl.kernel(
        _sc_affinity_body,
        out_type=jax.ShapeDtypeStruct((N * 8, N // 8), jnp.float32),
        mesh=mesh,
        scratch_types=[
            pltpu.VMEM((N,), jnp.int32),
            pltpu.VMEM((_ROWS,), jnp.int32),
            pltpu.VMEM((2, 128), jnp.int32),
            pltpu.VMEM((_ROWS * 8, N // 8), jnp.float32),
            pltpu.VMEM((_ROWS * 8, N // 8), jnp.float32),
            pltpu.SemaphoreType.DMA,
            pltpu.SemaphoreType.DMA,
        ],
        compiler_params=pltpu.CompilerParams(
            use_tc_tiling_on_sc=False, needs_layout_passes=False
        ),
    )
    return fn(y_true, lut_sub)


_RB = 128
_LUTB = 200


def _tc_retile_body(lut_ref, out_ref):
    x = lut_ref[...]
    x = jnp.concatenate(
        [x, jnp.zeros((_LUTB, N - M), jnp.float32)], axis=1)
    out_ref[...] = jnp.reshape(x, (_LUTB * 8, N // 8))


def _tc_retile(lookup):
    grid = M // _LUTB
    return pl.pallas_call(
        _tc_retile_body,
        grid=(grid,),
        in_specs=[pl.BlockSpec((_LUTB, M), lambda i: (i, 0))],
        out_specs=pl.BlockSpec((_LUTB * 8, N // 8), lambda i: (i, 0)),
        out_shape=jax.ShapeDtypeStruct((M * 8, N // 8), jnp.float32),
        compiler_params=pltpu.CompilerParams(
            dimension_semantics=("parallel",),
        ),
    )(lookup)


def _norm_rows(yp):
    colmask = lax.broadcasted_iota(jnp.int32, yp.shape, 1) < D_USE
    ypm = jnp.where(colmask, yp, 0.0)
    sq = jnp.sum(ypm * ypm, axis=1, keepdims=True)
    inv = 1.0 / jnp.maximum(jnp.sqrt(sq), EPS)
    return ypm * inv


def _tc_gram_body(ypb_ref, ypf_ref, g_ref):
    ynb = _norm_rows(ypb_ref[...])
    ynf = _norm_rows(ypf_ref[...])
    g_ref[...] = lax.dot_general(
        ynb, ynf, (((1,), (1,)), ((), ())),
        preferred_element_type=jnp.float32,
        precision=lax.Precision.HIGHEST,
    )


def _tc_gram(y_pred):
    grid = N // _RB
    return pl.pallas_call(
        _tc_gram_body,
        grid=(grid,),
        in_specs=[
            pl.BlockSpec((_RB, D), lambda i: (i, 0)),
            pl.BlockSpec((N, D), lambda i: (0, 0)),
        ],
        out_specs=pl.BlockSpec((_RB, N), lambda i: (i, 0)),
        out_shape=jax.ShapeDtypeStruct((N, N), jnp.float32),
        compiler_params=pltpu.CompilerParams(
            dimension_semantics=("parallel",),
        ),
    )(y_pred, y_pred)


_CB = 512


def _tc_combine_body(g_ref, a_ref, out_ref):
    i = pl.program_id(0)
    a = jnp.reshape(a_ref[...], (_CB, N))
    row = i * _CB + lax.broadcasted_iota(jnp.int32, (_CB, N), 0)
    col = lax.broadcasted_iota(jnp.int32, (_CB, N), 1)
    diff = jnp.where(col > row, jnp.abs(g_ref[...] - a), 0.0)
    s = jnp.sum(diff) * (1.0 / NPAIRS)

    @pl.when(i == 0)
    def _():
        out_ref[...] = jnp.zeros_like(out_ref)

    out_ref[...] += jnp.reshape(s, (1, 1))


def _tc_combine(g_mat, a_t):
    grid = N // _CB
    return pl.pallas_call(
        _tc_combine_body,
        grid=(grid,),
        in_specs=[
            pl.BlockSpec((_CB, N), lambda i: (i, 0)),
            pl.BlockSpec((_CB * 8, N // 8), lambda i: (i, 0)),
        ],
        out_specs=pl.BlockSpec((1, 1), lambda i: (0, 0)),
        out_shape=jax.ShapeDtypeStruct((1, 1), jnp.float32),
        compiler_params=pltpu.CompilerParams(
            dimension_semantics=("arbitrary",),
        ),
    )(g_mat, a_t)


@jax.jit
def kernel(y_true, y_pred, lookup):
    yt = y_true.astype(jnp.int32)
    lut_sub = _tc_retile(lookup)
    a_t = _sc_affinity(yt, lut_sub)
    g_mat = _tc_gram(y_pred)
    return _tc_combine(g_mat, a_t)[0, 0]


# speedup vs baseline: 1.0321x; 1.0123x over previous
import functools

import jax
import jax.numpy as jnp
from jax import lax
from jax.experimental import pallas as pl
from jax.experimental.pallas import tpu as pltpu
from jax.experimental.pallas import tpu_sc as plsc

N = 1024
D = 64
D_USE = 48
M = 1000
MPAD = 1024
EPS = 1e-8
NPAIRS = N * (N - 1) // 2

_NC = 2
_NS = 16
_NW = _NC * _NS
_ROWS = N // _NW
_L = 16


def _sc_affinity_body(yt_hbm, lut_hbm, a_hbm, yt_v, idx_v, sidx_v, rows_v,
                      out_v, sem, sem2):
    wid = lax.axis_index("s") * _NC + lax.axis_index("c")
    base = pl.multiple_of(wid * _ROWS, _ROWS)
    pltpu.sync_copy(yt_hbm, yt_v)
    pltpu.sync_copy(yt_hbm.at[pl.ds(base, _ROWS)], idx_v)
    lane = lax.iota(jnp.int32, _L)
    for v in range(_ROWS * 8 // _L):
        j = lane + (v * _L)
        yt16 = plsc.load_gather(idx_v, [j >> 3])
        sidx_v[v // 8, pl.ds((v % 8) * _L, _L)] = (yt16 << 3) + (j & 7)
    cp0 = pltpu.async_copy(
        lut_hbm.at[sidx_v.at[0]], rows_v.at[pl.ds(0, 128)], sem)
    cp1 = pltpu.async_copy(
        lut_hbm.at[sidx_v.at[1]], rows_v.at[pl.ds(128, 128)], sem)
    cp0.wait()
    cp1.wait()

    def g_body(g, _):
        gbase = pl.multiple_of(g * _L, _L)
        col16 = yt_v[pl.ds(gbase, _L)]
        srow = col16 >> 7
        lcol = col16 & 127
        sub = g // 8
        off = pl.multiple_of((g % 8) * _L, _L)
        for r0 in range(0, _ROWS, 8):
            vals = [
                plsc.load_gather(rows_v, [srow + (r0 + k) * 8, lcol])
                for k in range(8)
            ]
            for k in range(8):
                out_v[(r0 + k) * 8 + sub, pl.ds(off, _L)] = vals[k]
        return 0

    lax.fori_loop(0, N // _L, g_body, 0)
    pltpu.sync_copy(out_v, a_hbm.at[pl.ds(base * 8, _ROWS * 8), :])


def _sc_affinity(y_true, lut_sub):
    mesh = plsc.VectorSubcoreMesh(core_axis_name="c", subcore_axis_name="s")
    fn = pl.kernel(
        _sc_affinity_body,
        out_type=jax.ShapeDtypeStruct((N * 8, N // 8), jnp.float32),
        mesh=mesh,
        scratch_types=[
            pltpu.VMEM((N,), jnp.int32),
            pltpu.VMEM((_ROWS,), jnp.int32),
            pltpu.VMEM((2, 128), jnp.int32),
            pltpu.VMEM((_ROWS * 8, N // 8), jnp.float32),
            pltpu.VMEM((_ROWS * 8, N // 8), jnp.float32),
            pltpu.SemaphoreType.DMA,
            pltpu.SemaphoreType.DMA,
        ],
        compiler_params=pltpu.CompilerParams(
            use_tc_tiling_on_sc=False, needs_layout_passes=False
        ),
    )
    return fn(y_true, lut_sub)


_RB = 128
_LUTB = 200


def _tc_retile_body(lut_ref, out_ref):
    x = lut_ref[...]
    x = jnp.concatenate(
        [x, jnp.zeros((_LUTB, N - M), jnp.float32)], axis=1)
    out_ref[...] = jnp.reshape(x, (_LUTB * 8, N // 8))


def _tc_retile(lookup):
    grid = M // _LUTB
    return pl.pallas_call(
        _tc_retile_body,
        grid=(grid,),
        in_specs=[pl.BlockSpec((_LUTB, M), lambda i: (i, 0))],
        out_specs=pl.BlockSpec((_LUTB * 8, N // 8), lambda i: (i, 0)),
        out_shape=jax.ShapeDtypeStruct((M * 8, N // 8), jnp.float32),
        compiler_params=pltpu.CompilerParams(
            dimension_semantics=("parallel",),
        ),
    )(lookup)


def _norm_rows(yp):
    colmask = lax.broadcasted_iota(jnp.int32, yp.shape, 1) < D_USE
    ypm = jnp.where(colmask, yp, 0.0)
    sq = jnp.sum(ypm * ypm, axis=1, keepdims=True)
    inv = 1.0 / jnp.maximum(jnp.sqrt(sq), EPS)
    return ypm * inv


def _tc_gram_body(ypb_ref, ypf_ref, g_ref):
    ynb = _norm_rows(ypb_ref[...])
    ynf = _norm_rows(ypf_ref[...])
    g_ref[...] = lax.dot_general(
        ynb, ynf, (((1,), (1,)), ((), ())),
        preferred_element_type=jnp.float32,
        precision=lax.Precision.HIGHEST,
    ).astype(jnp.bfloat16)


def _tc_gram(y_pred):
    grid = N // _RB
    return pl.pallas_call(
        _tc_gram_body,
        grid=(grid,),
        in_specs=[
            pl.BlockSpec((_RB, D), lambda i: (i, 0)),
            pl.BlockSpec((N, D), lambda i: (0, 0)),
        ],
        out_specs=pl.BlockSpec((_RB, N), lambda i: (i, 0)),
        out_shape=jax.ShapeDtypeStruct((N, N), jnp.bfloat16),
        compiler_params=pltpu.CompilerParams(
            dimension_semantics=("parallel",),
        ),
    )(y_pred, y_pred)


_CB = 512


def _tc_combine_body(g_ref, a_ref, out_ref):
    i = pl.program_id(0)
    a = jnp.reshape(a_ref[...], (_CB, N))
    row = i * _CB + lax.broadcasted_iota(jnp.int32, (_CB, N), 0)
    col = lax.broadcasted_iota(jnp.int32, (_CB, N), 1)
    diff = jnp.where(col > row, jnp.abs(g_ref[...].astype(jnp.float32) - a), 0.0)
    s = jnp.sum(diff) * (1.0 / NPAIRS)

    @pl.when(i == 0)
    def _():
        out_ref[...] = jnp.zeros_like(out_ref)

    out_ref[...] += jnp.reshape(s, (1, 1))


def _tc_combine(g_mat, a_t):
    grid = N // _CB
    return pl.pallas_call(
        _tc_combine_body,
        grid=(grid,),
        in_specs=[
            pl.BlockSpec((_CB, N), lambda i: (i, 0)),
            pl.BlockSpec((_CB * 8, N // 8), lambda i: (i, 0)),
        ],
        out_specs=pl.BlockSpec((1, 1), lambda i: (0, 0)),
        out_shape=jax.ShapeDtypeStruct((1, 1), jnp.float32),
        compiler_params=pltpu.CompilerParams(
            dimension_semantics=("arbitrary",),
        ),
    )(g_mat, a_t)


@jax.jit
def kernel(y_true, y_pred, lookup):
    yt = y_true.astype(jnp.int32)
    lut_sub = _tc_retile(lookup)
    a_t = _sc_affinity(yt, lut_sub)
    g_mat = _tc_gram(y_pred)
    return _tc_combine(g_mat, a_t)[0, 0]


# speedup vs baseline: 1.0332x; 1.0010x over previous
import functools

import jax
import jax.numpy as jnp
from jax import lax
from jax.experimental import pallas as pl
from jax.experimental.pallas import tpu as pltpu
from jax.experimental.pallas import tpu_sc as plsc

N = 1024
D = 64
D_USE = 48
M = 1000
MPAD = 1024
EPS = 1e-8
NPAIRS = N * (N - 1) // 2

_NC = 2
_NS = 16
_NW = _NC * _NS
_ROWS = N // _NW
_L = 16


def _sc_affinity_body(yt_hbm, lut_hbm, a_hbm, yt_v, idx_v, sidx_v, rows_v,
                      out_v, sem, sem2):
    wid = lax.axis_index("s") * _NC + lax.axis_index("c")
    base = pl.multiple_of(wid * _ROWS, _ROWS)
    pltpu.sync_copy(yt_hbm, yt_v)
    pltpu.sync_copy(yt_hbm.at[pl.ds(base, _ROWS)], idx_v)
    lane = lax.iota(jnp.int32, _L)
    for v in range(_ROWS * 8 // _L):
        j = lane + (v * _L)
        yt16 = plsc.load_gather(idx_v, [j >> 3])
        sidx_v[v // 8, pl.ds((v % 8) * _L, _L)] = (yt16 << 3) + (j & 7)
    cp0 = pltpu.async_copy(
        lut_hbm.at[sidx_v.at[0]], rows_v.at[pl.ds(0, 128)], sem)
    cp1 = pltpu.async_copy(
        lut_hbm.at[sidx_v.at[1]], rows_v.at[pl.ds(128, 128)], sem)
    cp0.wait()
    cp1.wait()

    def oct_body(sub, _):
        gb = pl.multiple_of(sub * 128, 128)
        for gi in range(8):
            col16 = yt_v[pl.ds(gb + gi * _L, _L)]
            srow = col16 >> 7
            lcol = col16 & 127
            off = gi * _L
            for r0 in range(0, _ROWS, 8):
                vals = [
                    plsc.load_gather(rows_v, [srow + (r0 + k) * 8, lcol])
                    for k in range(8)
                ]
                for k in range(8):
                    out_v[(r0 + k) * 8 + sub, pl.ds(off, _L)] = vals[k]
        return 0

    lax.fori_loop(0, 8, oct_body, 0)
    pltpu.sync_copy(out_v, a_hbm.at[pl.ds(base * 8, _ROWS * 8), :])


def _sc_affinity(y_true, lut_sub):
    mesh = plsc.VectorSubcoreMesh(core_axis_name="c", subcore_axis_name="s")
    fn = pl.kernel(
        _sc_affinity_body,
        out_type=jax.ShapeDtypeStruct((N * 8, N // 8), jnp.float32),
        mesh=mesh,
        scratch_types=[
            pltpu.VMEM((N,), jnp.int32),
            pltpu.VMEM((_ROWS,), jnp.int32),
            pltpu.VMEM((2, 128), jnp.int32),
            pltpu.VMEM((_ROWS * 8, N // 8), jnp.float32),
            pltpu.VMEM((_ROWS * 8, N // 8), jnp.float32),
            pltpu.SemaphoreType.DMA,
            pltpu.SemaphoreType.DMA,
        ],
        compiler_params=pltpu.CompilerParams(
            use_tc_tiling_on_sc=False, needs_layout_passes=False
        ),
    )
    return fn(y_true, lut_sub)


_RB = 128
_LUTB = 200


def _tc_retile_body(lut_ref, out_ref):
    x = lut_ref[...]
    x = jnp.concatenate(
        [x, jnp.zeros((_LUTB, N - M), jnp.float32)], axis=1)
    out_ref[...] = jnp.reshape(x, (_LUTB * 8, N // 8))


def _tc_retile(lookup):
    grid = M // _LUTB
    return pl.pallas_call(
        _tc_retile_body,
        grid=(grid,),
        in_specs=[pl.BlockSpec((_LUTB, M), lambda i: (i, 0))],
        out_specs=pl.BlockSpec((_LUTB * 8, N // 8), lambda i: (i, 0)),
        out_shape=jax.ShapeDtypeStruct((M * 8, N // 8), jnp.float32),
        compiler_params=pltpu.CompilerParams(
            dimension_semantics=("parallel",),
        ),
    )(lookup)


def _norm_rows(yp):
    colmask = lax.broadcasted_iota(jnp.int32, yp.shape, 1) < D_USE
    ypm = jnp.where(colmask, yp, 0.0)
    sq = jnp.sum(ypm * ypm, axis=1, keepdims=True)
    inv = 1.0 / jnp.maximum(jnp.sqrt(sq), EPS)
    return ypm * inv


def _tc_gram_body(ypb_ref, ypf_ref, g_ref):
    ynb = _norm_rows(ypb_ref[...])
    ynf = _norm_rows(ypf_ref[...])
    g_ref[...] = lax.dot_general(
        ynb, ynf, (((1,), (1,)), ((), ())),
        preferred_element_type=jnp.float32,
        precision=lax.Precision.HIGHEST,
    ).astype(jnp.bfloat16)


def _tc_gram(y_pred):
    grid = N // _RB
    return pl.pallas_call(
        _tc_gram_body,
        grid=(grid,),
        in_specs=[
            pl.BlockSpec((_RB, D), lambda i: (i, 0)),
            pl.BlockSpec((N, D), lambda i: (0, 0)),
        ],
        out_specs=pl.BlockSpec((_RB, N), lambda i: (i, 0)),
        out_shape=jax.ShapeDtypeStruct((N, N), jnp.bfloat16),
        compiler_params=pltpu.CompilerParams(
            dimension_semantics=("parallel",),
        ),
    )(y_pred, y_pred)


_CB = 512


def _tc_combine_body(g_ref, a_ref, out_ref):
    i = pl.program_id(0)
    a = jnp.reshape(a_ref[...], (_CB, N))
    row = i * _CB + lax.broadcasted_iota(jnp.int32, (_CB, N), 0)
    col = lax.broadcasted_iota(jnp.int32, (_CB, N), 1)
    diff = jnp.where(col > row, jnp.abs(g_ref[...].astype(jnp.float32) - a), 0.0)
    s = jnp.sum(diff) * (1.0 / NPAIRS)

    @pl.when(i == 0)
    def _():
        out_ref[...] = jnp.zeros_like(out_ref)

    out_ref[...] += jnp.reshape(s, (1, 1))


def _tc_combine(g_mat, a_t):
    grid = N // _CB
    return pl.pallas_call(
        _tc_combine_body,
        grid=(grid,),
        in_specs=[
            pl.BlockSpec((_CB, N), lambda i: (i, 0)),
            pl.BlockSpec((_CB * 8, N // 8), lambda i: (i, 0)),
        ],
        out_specs=pl.BlockSpec((1, 1), lambda i: (0, 0)),
        out_shape=jax.ShapeDtypeStruct((1, 1), jnp.float32),
        compiler_params=pltpu.CompilerParams(
            dimension_semantics=("arbitrary",),
        ),
    )(g_mat, a_t)


@jax.jit
def kernel(y_true, y_pred, lookup):
    yt = y_true.astype(jnp.int32)
    lut_sub = _tc_retile(lookup)
    a_t = _sc_affinity(yt, lut_sub)
    g_mat = _tc_gram(y_pred)
    return _tc_combine(g_mat, a_t)[0, 0]
